# baseline (speedup 1.0000x reference)
import functools

import jax
import jax.numpy as jnp
from jax import lax
from jax.experimental import pallas as pl
from jax.experimental.pallas import tpu as pltpu
from jax.experimental.pallas import tpu_sc as plsc

L = 16
B = 128
N = 8192
NB1 = 4096
NBC = 256
NB2 = 1024
NW = 32
RPW = B // NW

INFKEY = 0x7F800000


def _sc_body(e_hbm, m_hbm, w1_hbm, b1_hbm, w2t_hbm, b2_hbm, out_hbm,
             ev, mv, kv, h1, h2, fv, w1v, b1v, w2tv, b2v):
    cid = lax.axis_index("c")
    sid = lax.axis_index("s")
    wid = sid * 2 + cid

    ones_i = jnp.ones((L,), jnp.int32)
    zero_i = jnp.zeros((L,), jnp.int32)
    zero_f = jnp.zeros((L,), jnp.float32)
    inf = jnp.float32(jnp.inf)

    pltpu.sync_copy(w1_hbm, w1v)
    pltpu.sync_copy(b1_hbm, b1v)
    pltpu.sync_copy(w2t_hbm, w2tv)
    pltpu.sync_copy(b2_hbm, b2v)

    for r in range(RPW):
        row = wid * RPW + r
        pltpu.sync_copy(e_hbm.at[row], ev)
        pltpu.sync_copy(m_hbm.at[row], mv)

        @plsc.parallel_loop(0, NB1 // L, unroll=4)
        def zero_body(i):
            h1[pl.ds(i * L, L)] = zero_i
            h2[pl.ds(i * L, L)] = zero_i

        @plsc.parallel_loop(0, N // L, unroll=4,
                            carry=(zero_f, zero_f, zero_f,
                                   jnp.full((L,), inf)))
        def scan1(i, c):
            nf, s1, s2, mn = c
            sl = pl.ds(i * L, L)
            e = ev[sl]
            m = jnp.minimum(jnp.maximum(mv[sl], 0.0), 1.0)
            keep = m > 0.0
            x = plsc.bitcast(e, jnp.int32)
            key = x ^ ((x >> 31) & 0x7FFFFFFF)
            key = jnp.where(keep, key, INFKEY)
            kv[sl] = key
            b1 = (key >> 20) + 2048
            plsc.addupdate_scatter(
                h1, [((b1 & 15) << 8) + (b1 >> 4)], ones_i)
            nf = nf + m
            s1 = s1 + e * m
            s2 = s2 + e * e * m
            mn = jnp.minimum(mn, jnp.where(keep, e, inf))
            return nf, s1, s2, mn
        nf, s1, s2, mn = scan1
        nf_s = jnp.sum(nf)
        s1_s = jnp.sum(s1)
        s2_s = jnp.sum(s2)
        mn_s = jnp.min(mn)

        def ffloor(x):
            i = x.astype(jnp.int32)
            return i - (i.astype(jnp.float32) > x).astype(jnp.int32)

        n_i = jnp.maximum(ffloor(nf_s), 1)
        nm1 = n_i - 1

        def rk(q):
            return jnp.clip(ffloor((nf_s - 1.0) * q), 0, nm1)
        kk = (rk(0.25), rk(0.5), rk(0.75), nm1)

        def findc(i, c):
            tot = h1[pl.ds(i * L, L)]
            for u in range(1, 16):
                tot = tot + h1[pl.ds(u * NBC + i * L, L)]
            cum = plsc.cumsum(tot) + c[0]
            out = [jnp.broadcast_to(jnp.max(cum), (L,))]
            for t in range(4):
                lt = cum < (kk[t] + 1)
                out.append(c[1 + t] + plsc.all_reduce_population_count(lt))
                out.append(jnp.maximum(c[5 + t], jnp.where(lt, cum, 0)))
            return (out[0], out[1], out[3], out[5], out[7],
                    out[2], out[4], out[6], out[8])
        res = lax.fori_loop(0, NBC // L, findc, (zero_i,) * 9)
        j_s, base_s = [], []
        for t in range(4):
            cc = jnp.clip(jnp.max(res[1 + t]), 0, NBC - 1)
            cb = jnp.max(res[5 + t])
            fine = plsc.load_gather(h1, [lax.iota(jnp.int32, L) * NBC + cc])
            cum = plsc.cumsum(fine) + cb
            lt = cum < (kk[t] + 1)
            j = cc * L + jnp.max(plsc.all_reduce_population_count(lt))
            j_s.append(jnp.clip(j, 0, NB1 - 1))
            base_s.append(jnp.maximum(cb, jnp.max(jnp.where(lt, cum, 0))))

        @plsc.parallel_loop(0, N // L, unroll=4)
        def hist2(i):
            key = kv[pl.ds(i * L, L)]
            b1 = (key >> 20) + 2048
            sub = (key >> 10) & (NB2 - 1)
            eq = [b1 == j_s[t] for t in range(4)]
            sel = jnp.where(eq[0], 0,
                            jnp.where(eq[1], 1, jnp.where(eq[2], 2, 3)))
            msk = (eq[0] | eq[1]) | (eq[2] | eq[3])
            plsc.addupdate_scatter(
                h2, [sub + (sel << 10)], ones_i, mask=msk)

        own = [jnp.int32(0)]
        own.append(jnp.where(j_s[1] == j_s[0], 0, 1))
        own.append(jnp.where(j_s[2] == j_s[0], 0,
                             jnp.where(j_s[2] == j_s[1], 1, 2)))
        own.append(jnp.where(j_s[3] == j_s[0], 0,
                             jnp.where(j_s[3] == j_s[1], 1,
                                       jnp.where(j_s[3] == j_s[2], 2, 3))))

        def find2(i, c):
            out = []
            for t in range(4):
                cum = plsc.cumsum(h2[pl.ds(own[t] * NB2 + i * L, L)]) + c[t]
                out.append(jnp.broadcast_to(jnp.max(cum), (L,)))
                lt = cum < (kk[t] - base_s[t] + 1)
                out.append(c[4 + t] + plsc.all_reduce_population_count(lt))
            return (out[0], out[2], out[4], out[6],
                    out[1], out[3], out[5], out[7])
        res2 = lax.fori_loop(0, NB2 // L, find2, (zero_i,) * 8)

        qv = []
        for t in range(4):
            j2 = jnp.clip(jnp.max(res2[4 + t]), 0, NB2 - 1)
            keyq = ((j_s[t] - 2048) << 20) + (j2 << 10) + (NB2 // 2)
            xq = keyq ^ ((keyq >> 31) & 0x7FFFFFFF)
            qv.append(lax.bitcast_convert_type(xq, jnp.float32))

        d = jnp.broadcast_to(nf_s + 1e-8, (L,))
        rcp = jnp.full((L,), 1.0, jnp.float32) / d
        meanv = jnp.broadcast_to(s1_s, (L,)) * rcp
        varv = (jnp.broadcast_to(s2_s, (L,))
                - 2.0 * meanv * jnp.broadcast_to(s1_s, (L,))
                + meanv * meanv * jnp.broadcast_to(nf_s, (L,))) * rcp
        a = varv + 1e-6
        y = plsc.bitcast(0x5F3759DF - (plsc.bitcast(a, jnp.int32) >> 1),
                         jnp.float32)
        for _ in range(3):
            y = y * (1.5 - 0.5 * a * y * y)
        stdv = a * y
        feats = (meanv, stdv, jnp.broadcast_to(mn_s, (L,)),
                 jnp.broadcast_to(qv[3], (L,)), jnp.broadcast_to(qv[0], (L,)),
                 jnp.broadcast_to(qv[1], (L,)), jnp.broadcast_to(qv[2], (L,)))
        h = b1v[...]
        for t in range(7):
            h = h + feats[t] * w1v[pl.ds(t * L, L)]
        h = jnp.maximum(h, 0.0)
        io = lax.iota(jnp.int32, L)
        f = zero_f
        for o in range(8):
            dot = jnp.sum(h * w2tv[pl.ds(o * L, L)])
            f = jnp.where(io == o, dot, f)
        fv[...] = f + b2v[...]
        pltpu.sync_copy(fv, out_hbm.at[row])


_sc_moments = functools.partial(
    pl.kernel,
    mesh=plsc.VectorSubcoreMesh(core_axis_name="c", subcore_axis_name="s"),
    out_type=jax.ShapeDtypeStruct((B, L), jnp.float32),
    compiler_params=pltpu.CompilerParams(needs_layout_passes=False),
    scratch_types=[
        pltpu.VMEM((N,), jnp.float32),
        pltpu.VMEM((N,), jnp.float32),
        pltpu.VMEM((N,), jnp.int32),
        pltpu.VMEM((NB1,), jnp.int32),
        pltpu.VMEM((4 * NB2,), jnp.int32),
        pltpu.VMEM((L,), jnp.float32),
        pltpu.VMEM((7 * L,), jnp.float32),
        pltpu.VMEM((L,), jnp.float32),
        pltpu.VMEM((8 * L,), jnp.float32),
        pltpu.VMEM((L,), jnp.float32),
    ],
)(_sc_body)


def kernel(energies, mask, W1, b1, W2, b2):
    w1f = W1.reshape(7 * 16)
    w2tf = W2.T.reshape(8 * 16)
    b2p = jnp.concatenate([b2, jnp.zeros((8,), b2.dtype)])
    out16 = _sc_moments(energies, mask, w1f, b1, w2tf, b2p)
    return out16[:, :8]


# speedup vs baseline: 1.1479x; 1.1479x over previous
import functools

import jax
import jax.numpy as jnp
from jax import lax
from jax.experimental import pallas as pl
from jax.experimental.pallas import tpu as pltpu
from jax.experimental.pallas import tpu_sc as plsc

L = 16
B = 128
N = 8192
NB1 = 4096
NBC = 256
NB2 = 1024
NW = 32
RPW = B // NW

INFKEY = 0x7F800000


def _sc_body(e_hbm, m_hbm, out_hbm, ev0, mv0, ev1, mv1, kv, h1, h2, fv,
             sem0, sem1):
    cid = lax.axis_index("c")
    sid = lax.axis_index("s")
    wid = sid * 2 + cid

    ones_i = jnp.ones((L,), jnp.int32)
    zero_i = jnp.zeros((L,), jnp.int32)
    zero_f = jnp.zeros((L,), jnp.float32)
    inf = jnp.float32(jnp.inf)

    bufs = ((ev0, mv0), (ev1, mv1))
    sems = (sem0, sem1)

    def start(rw, b):
        return (pltpu.async_copy(e_hbm.at[rw], bufs[b][0], sems[b]),
                pltpu.async_copy(m_hbm.at[rw], bufs[b][1], sems[b]))

    base = wid * RPW
    pend = [start(base, 0), None]
    for r in range(RPW):
        row = base + r
        b = r % 2
        ev, mv = bufs[b]
        for cp in pend[b]:
            cp.wait()
        if r + 1 < RPW:
            pend[1 - b] = start(row + 1, 1 - b)

        @plsc.parallel_loop(0, NB1 // L, unroll=4)
        def zero_body(i):
            h1[pl.ds(i * L, L)] = zero_i
            h2[pl.ds(i * L, L)] = zero_i

        @plsc.parallel_loop(0, N // L, unroll=4,
                            carry=(zero_f, zero_f, zero_f,
                                   jnp.full((L,), inf)))
        def scan1(i, c):
            nf, s1, s2, mn = c
            sl = pl.ds(i * L, L)
            e = ev[sl]
            m = jnp.minimum(jnp.maximum(mv[sl], 0.0), 1.0)
            keep = m > 0.0
            x = plsc.bitcast(e, jnp.int32)
            key = x ^ ((x >> 31) & 0x7FFFFFFF)
            key = jnp.where(keep, key, INFKEY)
            kv[sl] = key
            b1 = (key >> 20) + 2048
            plsc.addupdate_scatter(
                h1, [((b1 & 15) << 8) + (b1 >> 4)], ones_i)
            nf = nf + m
            s1 = s1 + e * m
            s2 = s2 + e * e * m
            mn = jnp.minimum(mn, jnp.where(keep, e, inf))
            return nf, s1, s2, mn
        nf, s1, s2, mn = scan1
        nf_s = jnp.sum(nf)
        s1_s = jnp.sum(s1)
        s2_s = jnp.sum(s2)
        mn_s = jnp.min(mn)

        def ffloor(x):
            i = x.astype(jnp.int32)
            return i - (i.astype(jnp.float32) > x).astype(jnp.int32)

        n_i = jnp.maximum(ffloor(nf_s), 1)
        nm1 = n_i - 1

        def rk(q):
            return jnp.clip(ffloor((nf_s - 1.0) * q), 0, nm1)
        kk = (rk(0.25), rk(0.5), rk(0.75), nm1)

        def findc(i, c):
            tot = h1[pl.ds(i * L, L)]
            for u in range(1, 16):
                tot = tot + h1[pl.ds(u * NBC + i * L, L)]
            cum = plsc.cumsum(tot) + c[0]
            out = [jnp.broadcast_to(jnp.max(cum), (L,))]
            for t in range(4):
                lt = cum < (kk[t] + 1)
                out.append(c[1 + t] + plsc.all_reduce_population_count(lt))
                out.append(jnp.maximum(c[5 + t], jnp.where(lt, cum, 0)))
            return (out[0], out[1], out[3], out[5], out[7],
                    out[2], out[4], out[6], out[8])
        res = lax.fori_loop(0, NBC // L, findc, (zero_i,) * 9)
        j_s, base_s = [], []
        for t in range(4):
            cc = jnp.clip(jnp.max(res[1 + t]), 0, NBC - 1)
            cb = jnp.max(res[5 + t])
            fine = plsc.load_gather(h1, [lax.iota(jnp.int32, L) * NBC + cc])
            cum = plsc.cumsum(fine) + cb
            lt = cum < (kk[t] + 1)
            j = cc * L + jnp.max(plsc.all_reduce_population_count(lt))
            j_s.append(jnp.clip(j, 0, NB1 - 1))
            base_s.append(jnp.maximum(cb, jnp.max(jnp.where(lt, cum, 0))))

        @plsc.parallel_loop(0, N // L, unroll=4)
        def hist2(i):
            key = kv[pl.ds(i * L, L)]
            b1 = (key >> 20) + 2048
            sub = (key >> 10) & (NB2 - 1)
            eq = [b1 == j_s[t] for t in range(4)]
            sel = jnp.where(eq[0], 0,
                            jnp.where(eq[1], 1, jnp.where(eq[2], 2, 3)))
            msk = (eq[0] | eq[1]) | (eq[2] | eq[3])
            plsc.addupdate_scatter(
                h2, [sub + (sel << 10)], ones_i, mask=msk)

        own = [jnp.int32(0)]
        own.append(jnp.where(j_s[1] == j_s[0], 0, 1))
        own.append(jnp.where(j_s[2] == j_s[0], 0,
                             jnp.where(j_s[2] == j_s[1], 1, 2)))
        own.append(jnp.where(j_s[3] == j_s[0], 0,
                             jnp.where(j_s[3] == j_s[1], 1,
                                       jnp.where(j_s[3] == j_s[2], 2, 3))))

        def find2(i, c):
            out = []
            for t in range(4):
                cum = plsc.cumsum(h2[pl.ds(own[t] * NB2 + i * L, L)]) + c[t]
                out.append(jnp.broadcast_to(jnp.max(cum), (L,)))
                lt = cum < (kk[t] - base_s[t] + 1)
                out.append(c[4 + t] + plsc.all_reduce_population_count(lt))
            return (out[0], out[2], out[4], out[6],
                    out[1], out[3], out[5], out[7])
        res2 = lax.fori_loop(0, NB2 // L, find2, (zero_i,) * 8)

        qv = []
        for t in range(4):
            j2 = jnp.clip(jnp.max(res2[4 + t]), 0, NB2 - 1)
            keyq = ((j_s[t] - 2048) << 20) + (j2 << 10) + (NB2 // 2)
            xq = keyq ^ ((keyq >> 31) & 0x7FFFFFFF)
            qv.append(lax.bitcast_convert_type(xq, jnp.float32))

        io = lax.iota(jnp.int32, L)
        f = zero_f
        for t, val in enumerate((nf_s, s1_s, s2_s, mn_s,
                                 qv[0], qv[1], qv[2], qv[3])):
            f = jnp.where(io == t, val, f)
        fv[...] = f
        pltpu.sync_copy(fv, out_hbm.at[row])


_sc_moments = functools.partial(
    pl.kernel,
    mesh=plsc.VectorSubcoreMesh(core_axis_name="c", subcore_axis_name="s"),
    out_type=jax.ShapeDtypeStruct((B, L), jnp.float32),
    compiler_params=pltpu.CompilerParams(needs_layout_passes=False),
    scratch_types=[
        pltpu.VMEM((N,), jnp.float32),
        pltpu.VMEM((N,), jnp.float32),
        pltpu.VMEM((N,), jnp.float32),
        pltpu.VMEM((N,), jnp.float32),
        pltpu.VMEM((N,), jnp.int32),
        pltpu.VMEM((NB1,), jnp.int32),
        pltpu.VMEM((4 * NB2,), jnp.int32),
        pltpu.VMEM((L,), jnp.float32),
        pltpu.SemaphoreType.DMA,
        pltpu.SemaphoreType.DMA,
    ],
)(_sc_body)


def _head_body(s_ref, w1_ref, b1_ref, w2_ref, b2_ref, o_ref):
    s = s_ref[...]
    nf = s[:, 0:1]
    s1 = s[:, 1:2]
    s2 = s[:, 2:3]
    d = nf + 1e-8
    mean = s1 / d
    var = (s2 - 2.0 * mean * s1 + mean * mean * nf) / d
    std = jnp.sqrt(var + 1e-6)
    feats = (mean, std, s[:, 3:4], s[:, 7:8], s[:, 4:5], s[:, 5:6], s[:, 6:7])
    w1 = w1_ref[...]
    h = jnp.broadcast_to(b1_ref[...], (B, 16))
    for t in range(7):
        h = h + feats[t] * w1[t:t + 1, :]
    h = jnp.maximum(h, 0.0)
    w2 = w2_ref[...]
    o = jnp.broadcast_to(b2_ref[...], (B, 8))
    for t in range(16):
        o = o + h[:, t:t + 1] * w2[t:t + 1, :]
    o_ref[...] = o


def kernel(energies, mask, W1, b1, W2, b2):
    feats16 = _sc_moments(energies, mask)
    W1p = jnp.concatenate([W1, jnp.zeros((1, 16), W1.dtype)], axis=0)
    return pl.pallas_call(
        _head_body,
        out_shape=jax.ShapeDtypeStruct((B, 8), jnp.float32),
    )(feats16, W1p, b1.reshape(1, 16), W2, b2.reshape(1, 8))
